# baseline (speedup 1.0000x reference)
import functools

import jax
import jax.numpy as jnp
from jax import lax
from jax.experimental import pallas as pl
from jax.experimental.pallas import tpu as pltpu, tpu_sc as plsc

N_NODES = 10000
N_PAD = 10240
IN_DIM = 128
HID_DIM = 256
OUT_DIM = 128
N_EDGES = 320000
E_PAD = 327680
PAD_NODE = N_NODES

NC, NS = 2, 16
NW = NC * NS
E_W = E_PAD // NW
K = 80
CH = E_W // K
IT = CH // 8
ROWS_T = N_PAD // NS

BN = 1024
GRID = N_PAD // BN

_mesh = plsc.VectorSubcoreMesh(core_axis_name="c", subcore_axis_name="s")


@functools.partial(
    pl.kernel,
    out_type=jax.ShapeDtypeStruct((NW, N_PAD), jnp.float32),
    mesh=_mesh,
    scratch_types=[
        pltpu.VMEM((E_W,), jnp.int32),
        pltpu.VMEM((N_PAD,), jnp.float32),
    ],
    compiler_params=pltpu.CompilerParams(needs_layout_passes=False),
)
def _deg_kernel(dst_hbm, out_hbm, idx_v, deg_v):
    c = lax.axis_index("c")
    s = lax.axis_index("s")
    wid = c * NS + s
    pltpu.sync_copy(dst_hbm.at[pl.ds(wid * E_W, E_W)], idx_v)

    def zero(i, _):
        deg_v[pl.ds(i * 16, 16)] = jnp.zeros((16,), jnp.float32)
        return 0
    lax.fori_loop(0, N_PAD // 16, zero, 0)

    ones = jnp.ones((16,), jnp.float32)

    def step(i, _):
        idx = idx_v[pl.ds(i * 16, 16)]
        plsc.addupdate_scatter(deg_v, [idx], ones)
        return 0
    lax.fori_loop(0, E_W // 16, step, 0)
    pltpu.sync_copy(deg_v, out_hbm.at[wid])


@functools.partial(
    pl.kernel,
    out_type=jax.ShapeDtypeStruct((NC, N_PAD, IN_DIM), jnp.float32),
    mesh=_mesh,
    scratch_types=[
        pltpu.VMEM((4, K), jnp.int32),
        pltpu.VMEM((4, K), jnp.int32),
        pltpu.VMEM((4, K), jnp.int32),
        pltpu.VMEM((4, K), jnp.int32),
        pltpu.VMEM((K, IN_DIM), jnp.float32),
        pltpu.VMEM((K, IN_DIM), jnp.float32),
        pltpu.VMEM((K, IN_DIM), jnp.float32),
        pltpu.VMEM((K, IN_DIM), jnp.float32),
        pltpu.VMEM_SHARED((N_PAD, IN_DIM), jnp.float32),
        pltpu.SemaphoreType.DMA,
        pltpu.SemaphoreType.DMA,
        pltpu.SemaphoreType.DMA,
        pltpu.SemaphoreType.DMA,
        pltpu.SemaphoreType.DMA,
        pltpu.SemaphoreType.DMA,
    ],
)
def _prop_kernel(xs_hbm, src_hbm, dst_hbm, zrows_hbm, out_hbm,
                 xs_i, xd_i, ys_i, yd_i, a0, a1, b0, b1, acc_shared,
                 sga, sgb, ssa, ssb, six, siy):
    c = lax.axis_index("c")
    s = lax.axis_index("s")
    wid = c * NS + s
    ibase = wid * CH

    def gather(idx_ref, row, buf, sem):
        pltpu.async_copy(xs_hbm.at[idx_ref.at[row]], buf, sem)

    def scatter(idx_ref, row, buf, sem):
        pltpu.async_copy(buf, acc_shared.at[idx_ref.at[row]], sem, add=True)

    def drain_rows(buf, sem):
        pltpu.make_async_copy(xs_hbm.at[pl.ds(0, K)], buf, sem).wait()

    def load_idx(group, s_ref, d_ref, sem):
        pltpu.async_copy(src_hbm.at[pl.ds(ibase + 4 * group, 4)], s_ref, sem)
        pltpu.async_copy(dst_hbm.at[pl.ds(ibase + 4 * group, 4)], d_ref, sem)

    def drain_idx(s_ref, d_ref, sem):
        pltpu.make_async_copy(src_hbm.at[pl.ds(0, 4)], s_ref, sem).wait()
        pltpu.make_async_copy(src_hbm.at[pl.ds(0, 4)], d_ref, sem).wait()

    pltpu.sync_copy(src_hbm.at[pl.ds(ibase, 4)], xs_i)
    pltpu.sync_copy(dst_hbm.at[pl.ds(ibase, 4)], xd_i)
    load_idx(1, ys_i, yd_i, siy)
    pltpu.sync_copy(zrows_hbm, acc_shared.at[pl.ds(s * ROWS_T, ROWS_T)])
    plsc.subcore_barrier()
    gather(xs_i, 0, a0, sga)
    gather(xs_i, 1, a1, sga)
    gather(xs_i, 2, b0, sgb)
    gather(xs_i, 3, b1, sgb)

    def step(m, _):
        drain_rows(a0, sga)
        drain_rows(a1, sga)
        scatter(xd_i, 0, a0, ssa)
        scatter(xd_i, 1, a1, ssa)
        drain_rows(a0, ssa)
        drain_rows(a1, ssa)
        drain_idx(ys_i, yd_i, siy)
        gather(ys_i, 0, a0, sga)
        gather(ys_i, 1, a1, sga)
        drain_rows(b0, sgb)
        drain_rows(b1, sgb)
        scatter(xd_i, 2, b0, ssb)
        scatter(xd_i, 3, b1, ssb)
        drain_rows(b0, ssb)
        drain_rows(b1, ssb)
        load_idx(2 * m + 2, xs_i, xd_i, six)
        gather(ys_i, 2, b0, sgb)
        gather(ys_i, 3, b1, sgb)
        drain_rows(a0, sga)
        drain_rows(a1, sga)
        scatter(yd_i, 0, a0, ssa)
        scatter(yd_i, 1, a1, ssa)
        drain_rows(a0, ssa)
        drain_rows(a1, ssa)
        drain_idx(xs_i, xd_i, six)
        gather(xs_i, 0, a0, sga)
        gather(xs_i, 1, a1, sga)
        drain_rows(b0, sgb)
        drain_rows(b1, sgb)
        scatter(yd_i, 2, b0, ssb)
        scatter(yd_i, 3, b1, ssb)
        drain_rows(b0, ssb)
        drain_rows(b1, ssb)
        load_idx(2 * m + 3, ys_i, yd_i, siy)
        gather(xs_i, 2, b0, sgb)
        gather(xs_i, 3, b1, sgb)
        return 0
    lax.fori_loop(0, IT, step, 0)

    drain_rows(a0, sga)
    drain_rows(a1, sga)
    drain_rows(b0, sgb)
    drain_rows(b1, sgb)
    drain_idx(ys_i, yd_i, siy)

    plsc.subcore_barrier()
    pltpu.sync_copy(acc_shared.at[pl.ds(s * ROWS_T, ROWS_T)],
                    out_hbm.at[c, pl.ds(s * ROWS_T, ROWS_T)])


def _scale_body(dp_ref, x_ref, xs_ref, disb_ref):
    dp = dp_ref[...]
    ones = jnp.ones((NW, IN_DIM), jnp.float32)
    degb = lax.dot_general(dp, ones, (((0,), (0,)), ((), ())),
                           preferred_element_type=jnp.float32) + 1.0
    disb = lax.rsqrt(degb)
    disb_ref[...] = disb
    xs_ref[...] = x_ref[...] * disb


def _scale_kernel(deg_part, x_pad):
    return pl.pallas_call(
        _scale_body,
        grid=(GRID,),
        in_specs=[
            pl.BlockSpec((NW, BN), lambda i: (0, i)),
            pl.BlockSpec((BN, IN_DIM), lambda i: (i, 0)),
        ],
        out_specs=[
            pl.BlockSpec((BN, IN_DIM), lambda i: (i, 0)),
            pl.BlockSpec((BN, IN_DIM), lambda i: (i, 0)),
        ],
        out_shape=[
            jax.ShapeDtypeStruct((N_PAD, IN_DIM), jnp.float32),
            jax.ShapeDtypeStruct((N_PAD, IN_DIM), jnp.float32),
        ],
    )(deg_part, x_pad)


def _l1_body(s1_ref, x_ref, disb_ref, w1_ref, b1_ref, w2_ref, g_ref, gs_ref):
    s1 = s1_ref[...]
    disb = disb_ref[...]
    p1 = disb * (s1[0] + s1[1]) + disb * disb * x_ref[...]
    h = jnp.dot(p1, w1_ref[...], preferred_element_type=jnp.float32)
    h = jnp.maximum(h + b1_ref[...], 0.0)
    g = jnp.dot(h, w2_ref[...], preferred_element_type=jnp.float32)
    g_ref[...] = g
    gs_ref[...] = disb * g


def _l1_kernel(S1, x_pad, disb, W1, b1, W2):
    return pl.pallas_call(
        _l1_body,
        grid=(GRID,),
        in_specs=[
            pl.BlockSpec((NC, BN, IN_DIM), lambda i: (0, i, 0)),
            pl.BlockSpec((BN, IN_DIM), lambda i: (i, 0)),
            pl.BlockSpec((BN, IN_DIM), lambda i: (i, 0)),
            pl.BlockSpec((IN_DIM, HID_DIM), lambda i: (0, 0)),
            pl.BlockSpec((1, HID_DIM), lambda i: (0, 0)),
            pl.BlockSpec((HID_DIM, OUT_DIM), lambda i: (0, 0)),
        ],
        out_specs=[
            pl.BlockSpec((BN, OUT_DIM), lambda i: (i, 0)),
            pl.BlockSpec((BN, OUT_DIM), lambda i: (i, 0)),
        ],
        out_shape=[
            jax.ShapeDtypeStruct((N_PAD, OUT_DIM), jnp.float32),
            jax.ShapeDtypeStruct((N_PAD, OUT_DIM), jnp.float32),
        ],
    )(S1, x_pad, disb, W1, b1, W2)


def _l2_body(s2_ref, g_ref, disb_ref, b2_ref, out_ref):
    s2 = s2_ref[...]
    disb = disb_ref[...]
    p2 = disb * (s2[0] + s2[1]) + disb * disb * g_ref[...]
    out_ref[...] = jnp.maximum(p2 + b2_ref[...], 0.0)


def _l2_kernel(S2, g, disb, b2):
    return pl.pallas_call(
        _l2_body,
        grid=(GRID,),
        in_specs=[
            pl.BlockSpec((NC, BN, OUT_DIM), lambda i: (0, i, 0)),
            pl.BlockSpec((BN, OUT_DIM), lambda i: (i, 0)),
            pl.BlockSpec((BN, OUT_DIM), lambda i: (i, 0)),
            pl.BlockSpec((1, OUT_DIM), lambda i: (0, 0)),
        ],
        out_specs=pl.BlockSpec((BN, OUT_DIM), lambda i: (i, 0)),
        out_shape=jax.ShapeDtypeStruct((N_PAD, OUT_DIM), jnp.float32),
    )(S2, g, disb, b2)


def kernel(x, edge_index, W1, b1, W2, b2):
    src = edge_index[0].astype(jnp.int32)
    dst = edge_index[1].astype(jnp.int32)
    pad = PAD_NODE + jnp.arange(E_PAD - N_EDGES, dtype=jnp.int32) % (
        N_PAD - N_NODES)
    extra = jnp.full((8 * K,), PAD_NODE, jnp.int32)
    src_p = jnp.concatenate([src, pad, extra]).reshape(NW * CH + 8, K)
    dst_p = jnp.concatenate([dst, pad, extra]).reshape(NW * CH + 8, K)
    dst_flat = jnp.concatenate([dst, pad])
    x_pad = jnp.pad(x, ((0, N_PAD - N_NODES), (0, 0)))
    zrows = jnp.zeros((ROWS_T, IN_DIM), jnp.float32)

    deg_part = _deg_kernel(dst_flat)
    xs, disb = _scale_kernel(deg_part, x_pad)
    S1 = _prop_kernel(xs, src_p, dst_p, zrows)
    g, gs = _l1_kernel(S1, x_pad, disb, W1,
                       b1.reshape(1, HID_DIM), W2)
    S2 = _prop_kernel(gs, src_p, dst_p, zrows)
    out = _l2_kernel(S2, g, disb, b2.reshape(1, OUT_DIM))
    return out[:N_NODES]


# speedup vs baseline: 1.0209x; 1.0209x over previous
import functools

import jax
import jax.numpy as jnp
from jax import lax
from jax.experimental import pallas as pl
from jax.experimental.pallas import tpu as pltpu, tpu_sc as plsc

N_NODES = 10000
N_PAD = 10240
IN_DIM = 128
HID_DIM = 256
OUT_DIM = 128
N_EDGES = 320000
E_PAD = 327680
PAD_NODE = N_NODES

NC, NS = 2, 16
NW = NC * NS
E_W = E_PAD // NW
K = 64
CH = E_W // K
IT = CH // 8
ROWS_T = N_PAD // NS

BN = 1024
GRID = N_PAD // BN

_mesh = plsc.VectorSubcoreMesh(core_axis_name="c", subcore_axis_name="s")


@functools.partial(
    pl.kernel,
    out_type=jax.ShapeDtypeStruct((NW, N_PAD), jnp.float32),
    mesh=_mesh,
    scratch_types=[
        pltpu.VMEM((E_W,), jnp.int32),
        pltpu.VMEM((N_PAD,), jnp.float32),
    ],
    compiler_params=pltpu.CompilerParams(needs_layout_passes=False),
)
def _deg_kernel(dst_hbm, out_hbm, idx_v, deg_v):
    c = lax.axis_index("c")
    s = lax.axis_index("s")
    wid = c * NS + s
    pltpu.sync_copy(dst_hbm.at[pl.ds(wid * E_W, E_W)], idx_v)

    def zero(i, _):
        deg_v[pl.ds(i * 16, 16)] = jnp.zeros((16,), jnp.float32)
        return 0
    lax.fori_loop(0, N_PAD // 16, zero, 0)

    ones = jnp.ones((16,), jnp.float32)

    def step(i, _):
        idx = idx_v[pl.ds(i * 16, 16)]
        plsc.addupdate_scatter(deg_v, [idx], ones)
        return 0
    lax.fori_loop(0, E_W // 16, step, 0)
    pltpu.sync_copy(deg_v, out_hbm.at[wid])


@functools.partial(
    pl.kernel,
    out_type=jax.ShapeDtypeStruct((NC, N_PAD, IN_DIM), jnp.float32),
    mesh=_mesh,
    scratch_types=[
        pltpu.VMEM((4, K), jnp.int32),
        pltpu.VMEM((4, K), jnp.int32),
        pltpu.VMEM((4, K), jnp.int32),
        pltpu.VMEM((4, K), jnp.int32),
        pltpu.VMEM((K, IN_DIM), jnp.float32),
        pltpu.VMEM((K, IN_DIM), jnp.float32),
        pltpu.VMEM((K, IN_DIM), jnp.float32),
        pltpu.VMEM((K, IN_DIM), jnp.float32),
        pltpu.VMEM_SHARED((N_PAD, IN_DIM), jnp.float32),
        pltpu.SemaphoreType.DMA,
        pltpu.SemaphoreType.DMA,
        pltpu.SemaphoreType.DMA,
        pltpu.SemaphoreType.DMA,
        pltpu.SemaphoreType.DMA,
        pltpu.SemaphoreType.DMA,
    ],
)
def _prop_kernel(xs_hbm, src_hbm, dst_hbm, zrows_hbm, out_hbm,
                 xs_i, xd_i, ys_i, yd_i, a0, a1, b0, b1, acc_shared,
                 sga, sgb, ssa, ssb, six, siy):
    c = lax.axis_index("c")
    s = lax.axis_index("s")
    wid = c * NS + s
    ibase = wid * CH

    def gather(idx_ref, row, buf, sem):
        pltpu.async_copy(xs_hbm.at[idx_ref.at[row]], buf, sem)

    def scatter(idx_ref, row, buf, sem):
        pltpu.async_copy(buf, acc_shared.at[idx_ref.at[row]], sem, add=True)

    def drain_rows(buf, sem):
        pltpu.make_async_copy(xs_hbm.at[pl.ds(0, K)], buf, sem).wait()

    def load_idx(group, s_ref, d_ref, sem):
        pltpu.async_copy(src_hbm.at[pl.ds(ibase + 4 * group, 4)], s_ref, sem)
        pltpu.async_copy(dst_hbm.at[pl.ds(ibase + 4 * group, 4)], d_ref, sem)

    def drain_idx(s_ref, d_ref, sem):
        pltpu.make_async_copy(src_hbm.at[pl.ds(0, 4)], s_ref, sem).wait()
        pltpu.make_async_copy(src_hbm.at[pl.ds(0, 4)], d_ref, sem).wait()

    pltpu.sync_copy(src_hbm.at[pl.ds(ibase, 4)], xs_i)
    pltpu.sync_copy(dst_hbm.at[pl.ds(ibase, 4)], xd_i)
    load_idx(1, ys_i, yd_i, siy)
    pltpu.sync_copy(zrows_hbm, acc_shared.at[pl.ds(s * ROWS_T, ROWS_T)])
    plsc.subcore_barrier()
    gather(xs_i, 0, a0, sga)
    gather(xs_i, 1, a1, sga)
    gather(xs_i, 2, b0, sgb)
    gather(xs_i, 3, b1, sgb)

    def step(m, _):
        drain_rows(a0, sga)
        drain_rows(a1, sga)
        scatter(xd_i, 0, a0, ssa)
        scatter(xd_i, 1, a1, ssa)
        drain_rows(a0, ssa)
        drain_rows(a1, ssa)
        drain_idx(ys_i, yd_i, siy)
        gather(ys_i, 0, a0, sga)
        gather(ys_i, 1, a1, sga)
        drain_rows(b0, sgb)
        drain_rows(b1, sgb)
        scatter(xd_i, 2, b0, ssb)
        scatter(xd_i, 3, b1, ssb)
        drain_rows(b0, ssb)
        drain_rows(b1, ssb)
        load_idx(2 * m + 2, xs_i, xd_i, six)
        gather(ys_i, 2, b0, sgb)
        gather(ys_i, 3, b1, sgb)
        drain_rows(a0, sga)
        drain_rows(a1, sga)
        scatter(yd_i, 0, a0, ssa)
        scatter(yd_i, 1, a1, ssa)
        drain_rows(a0, ssa)
        drain_rows(a1, ssa)
        drain_idx(xs_i, xd_i, six)
        gather(xs_i, 0, a0, sga)
        gather(xs_i, 1, a1, sga)
        drain_rows(b0, sgb)
        drain_rows(b1, sgb)
        scatter(yd_i, 2, b0, ssb)
        scatter(yd_i, 3, b1, ssb)
        drain_rows(b0, ssb)
        drain_rows(b1, ssb)
        load_idx(2 * m + 3, ys_i, yd_i, siy)
        gather(xs_i, 2, b0, sgb)
        gather(xs_i, 3, b1, sgb)
        return 0
    lax.fori_loop(0, IT, step, 0)

    drain_rows(a0, sga)
    drain_rows(a1, sga)
    drain_rows(b0, sgb)
    drain_rows(b1, sgb)
    drain_idx(ys_i, yd_i, siy)

    plsc.subcore_barrier()
    pltpu.sync_copy(acc_shared.at[pl.ds(s * ROWS_T, ROWS_T)],
                    out_hbm.at[c, pl.ds(s * ROWS_T, ROWS_T)])


def _scale_body(dp_ref, x_ref, xs_ref, disb_ref):
    dp = dp_ref[...]
    ones = jnp.ones((NW, IN_DIM), jnp.float32)
    degb = lax.dot_general(dp, ones, (((0,), (0,)), ((), ())),
                           preferred_element_type=jnp.float32) + 1.0
    disb = lax.rsqrt(degb)
    disb_ref[...] = disb
    xs_ref[...] = x_ref[...] * disb


def _scale_kernel(deg_part, x_pad):
    return pl.pallas_call(
        _scale_body,
        grid=(GRID,),
        in_specs=[
            pl.BlockSpec((NW, BN), lambda i: (0, i)),
            pl.BlockSpec((BN, IN_DIM), lambda i: (i, 0)),
        ],
        out_specs=[
            pl.BlockSpec((BN, IN_DIM), lambda i: (i, 0)),
            pl.BlockSpec((BN, IN_DIM), lambda i: (i, 0)),
        ],
        out_shape=[
            jax.ShapeDtypeStruct((N_PAD, IN_DIM), jnp.float32),
            jax.ShapeDtypeStruct((N_PAD, IN_DIM), jnp.float32),
        ],
    )(deg_part, x_pad)


def _l1_body(s1_ref, x_ref, disb_ref, w1_ref, b1_ref, w2_ref, g_ref, gs_ref):
    s1 = s1_ref[...]
    disb = disb_ref[...]
    p1 = disb * (s1[0] + s1[1]) + disb * disb * x_ref[...]
    h = jnp.dot(p1, w1_ref[...], preferred_element_type=jnp.float32)
    h = jnp.maximum(h + b1_ref[...], 0.0)
    g = jnp.dot(h, w2_ref[...], preferred_element_type=jnp.float32)
    g_ref[...] = g
    gs_ref[...] = disb * g


def _l1_kernel(S1, x_pad, disb, W1, b1, W2):
    return pl.pallas_call(
        _l1_body,
        grid=(GRID,),
        in_specs=[
            pl.BlockSpec((NC, BN, IN_DIM), lambda i: (0, i, 0)),
            pl.BlockSpec((BN, IN_DIM), lambda i: (i, 0)),
            pl.BlockSpec((BN, IN_DIM), lambda i: (i, 0)),
            pl.BlockSpec((IN_DIM, HID_DIM), lambda i: (0, 0)),
            pl.BlockSpec((1, HID_DIM), lambda i: (0, 0)),
            pl.BlockSpec((HID_DIM, OUT_DIM), lambda i: (0, 0)),
        ],
        out_specs=[
            pl.BlockSpec((BN, OUT_DIM), lambda i: (i, 0)),
            pl.BlockSpec((BN, OUT_DIM), lambda i: (i, 0)),
        ],
        out_shape=[
            jax.ShapeDtypeStruct((N_PAD, OUT_DIM), jnp.float32),
            jax.ShapeDtypeStruct((N_PAD, OUT_DIM), jnp.float32),
        ],
    )(S1, x_pad, disb, W1, b1, W2)


def _l2_body(s2_ref, g_ref, disb_ref, b2_ref, out_ref):
    s2 = s2_ref[...]
    disb = disb_ref[...]
    p2 = disb * (s2[0] + s2[1]) + disb * disb * g_ref[...]
    out_ref[...] = jnp.maximum(p2 + b2_ref[...], 0.0)


def _l2_kernel(S2, g, disb, b2):
    bn = N_NODES // GRID
    return pl.pallas_call(
        _l2_body,
        grid=(GRID,),
        in_specs=[
            pl.BlockSpec((NC, bn, OUT_DIM), lambda i: (0, i, 0)),
            pl.BlockSpec((bn, OUT_DIM), lambda i: (i, 0)),
            pl.BlockSpec((bn, OUT_DIM), lambda i: (i, 0)),
            pl.BlockSpec((1, OUT_DIM), lambda i: (0, 0)),
        ],
        out_specs=pl.BlockSpec((bn, OUT_DIM), lambda i: (i, 0)),
        out_shape=jax.ShapeDtypeStruct((N_NODES, OUT_DIM), jnp.float32),
    )(S2, g, disb, b2)


def kernel(x, edge_index, W1, b1, W2, b2):
    src = edge_index[0].astype(jnp.int32)
    dst = edge_index[1].astype(jnp.int32)
    pad = PAD_NODE + jnp.arange(E_PAD - N_EDGES, dtype=jnp.int32) % (
        N_PAD - N_NODES)
    extra = jnp.full((8 * K,), PAD_NODE, jnp.int32)
    src_p = jnp.concatenate([src, pad, extra]).reshape(NW * CH + 8, K)
    dst_p = jnp.concatenate([dst, pad, extra]).reshape(NW * CH + 8, K)
    dst_flat = jnp.concatenate([dst, pad])
    x_pad = jnp.pad(x, ((0, N_PAD - N_NODES), (0, 0)))
    zrows = jnp.zeros((ROWS_T, IN_DIM), jnp.float32)

    deg_part = _deg_kernel(dst_flat)
    xs, disb = _scale_kernel(deg_part, x_pad)
    S1 = _prop_kernel(xs, src_p, dst_p, zrows)
    g, gs = _l1_kernel(S1, x_pad, disb, W1,
                       b1.reshape(1, HID_DIM), W2)
    S2 = _prop_kernel(gs, src_p, dst_p, zrows)
    return _l2_kernel(S2, g, disb, b2.reshape(1, OUT_DIM))
